# baseline (speedup 1.0000x reference)
import functools
import jax
import jax.numpy as jnp
from jax import lax
from jax.experimental import pallas as pl
from jax.experimental.pallas import tpu as pltpu
from jax.experimental.pallas import tpu_sc as plsc

_D = 2048
_B = 4
_QK_SCALE = 16 ** -0.5
_SUBSTEPS = 8
_TPR = 8
_CHUNK = _D // _TPR
_NCH = _CHUNK // 16
_F32 = jnp.float32


def _bcast(v, lane):
    return jnp.take_along_axis(v, jnp.full((16,), lane, jnp.int32), axis=0)


def _sc_body(x_hbm, ep_hbm, wq_hbm, wk_hbm, bq_hbm, bk_hbm, cst_hbm, out_hbm,
             ep_v, x_v, xs_v, acc_v, qe_v, ke_v, qet_v,
             wq_v, wk_v, bq_v, bk_v, cst_v, part_v, all_v, accum_sh):
    cid = lax.axis_index("c")
    sid = lax.axis_index("s")
    row_l = sid // _TPR
    p = sid % _TPR
    r = cid * 2 + row_l
    j0 = p * _CHUNK

    pltpu.sync_copy(x_hbm.at[r, pl.ds(j0, _CHUNK)], x_v)
    pltpu.sync_copy(ep_hbm.at[pl.ds(j0 * 16, _CHUNK * 16)], ep_v)
    pltpu.sync_copy(wq_hbm, wq_v)
    pltpu.sync_copy(wk_hbm, wk_v)
    pltpu.sync_copy(bq_hbm, bq_v)
    pltpu.sync_copy(bk_hbm, bk_v)
    pltpu.sync_copy(cst_hbm, cst_v)
    pltpu.sync_copy(x_v, out_hbm.at[0, r, pl.ds(j0, _CHUNK)])

    def prep_row(j, _):
        ep_row = ep_v[pl.ds(j * 16, 16)]
        qe_row = bq_v[...]
        ke_row = bk_v[...]
        for e in range(16):
            be = _bcast(ep_row, e)
            qe_row = qe_row + be * wq_v[pl.ds(e * 16, 16)]
            ke_row = ke_row + be * wk_v[pl.ds(e * 16, 16)]
        qe_v[pl.ds(j * 16, 16)] = qe_row
        ke_v[pl.ds(j * 16, 16)] = ke_row
        return 0

    lax.fori_loop(0, _CHUNK, prep_row, 0)

    def prep_t(c, _):
        rows = (c * 16 + lax.iota(jnp.int32, 16)) * 16
        for d in range(16):
            col = plsc.load_gather(qe_v, [rows + d])
            qet_v[d, pl.ds(c * 16, 16)] = col
        return 0

    lax.fori_loop(0, _NCH, prep_t, 0)

    def init_xs(c, _):
        sl = pl.ds(c * 16, 16)
        xs_v[sl] = x_v[sl]
        return 0

    lax.fori_loop(0, _NCH, init_xs, 0)
    zv = jnp.zeros((16,), _F32)
    for i in range(3, 8):
        part_v[pl.ds(i * 16, 16)] = zv

    wq0 = cst_v[pl.ds(0, 16)]
    wk0 = cst_v[pl.ds(16, 16)]
    hv = cst_v[pl.ds(32, 16)]

    def do_stage(stage_i):
        def chunk_a(c, carry):
            uk, uq, ssv = carry
            xc = xs_v[pl.ds(c * 16, 16)]
            ssv = ssv + xc * xc
            for l in range(16):
                bx = _bcast(xc, l)
                jj = (c * 16 + l) * 16
                uk = uk + bx * ke_v[pl.ds(jj, 16)]
                uq = uq + bx * qe_v[pl.ds(jj, 16)]
            return uk, uq, ssv

        z = jnp.zeros((16,), _F32)
        uk, uq, ssv = lax.fori_loop(0, _NCH, chunk_a, (z, z, z))
        part_v[pl.ds(0, 16)] = uk
        part_v[pl.ds(16, 16)] = uq
        part_v[pl.ds(32, 16)] = ssv

        pltpu.sync_copy(part_v, accum_sh.at[row_l, p])
        plsc.subcore_barrier()
        pltpu.sync_copy(accum_sh.at[row_l], all_v)

        ukt = all_v[0, pl.ds(0, 16)]
        uqt = all_v[0, pl.ds(16, 16)]
        ssw = all_v[0, pl.ds(32, 16)]
        for q in range(1, _TPR):
            ukt = ukt + all_v[q, pl.ds(0, 16)]
            uqt = uqt + all_v[q, pl.ds(16, 16)]
            ssw = ssw + all_v[q, pl.ds(32, 16)]
        ss = jnp.sum(ssw)
        m = wk0 * ss + ukt
        a = jnp.sum(wq0 * m)
        g = _QK_SCALE * (a * ss + jnp.sum(uqt * m))
        md = [_bcast(m, d) for d in range(16)]

        def chunk_d(c, _):
            sl = pl.ds(c * 16, 16)
            xsc = xs_v[sl]
            pf = md[0] * qet_v[0, sl]
            for d in range(1, 16):
                pf = pf + md[d] * qet_v[d, sl]
            fit = _QK_SCALE * (a * xsc + pf)
            kc = xsc * (fit - g)
            if stage_i == 0:
                acc_v[sl] = kc
                xs_v[sl] = x_v[sl] + (0.5 * hv) * kc
            elif stage_i == 1:
                acc_v[sl] = acc_v[sl] + 2.0 * kc
                xs_v[sl] = x_v[sl] + (0.5 * hv) * kc
            elif stage_i == 2:
                acc_v[sl] = acc_v[sl] + 2.0 * kc
                xs_v[sl] = x_v[sl] + hv * kc
            else:
                xn = x_v[sl] + (hv * (1.0 / 6.0)) * (acc_v[sl] + kc)
                x_v[sl] = xn
                xs_v[sl] = xn
            return 0

        lax.fori_loop(0, _NCH, chunk_d, 0)
        plsc.subcore_barrier()

    def step(_, carry):
        do_stage(0)
        do_stage(1)
        do_stage(2)
        do_stage(3)
        return carry

    lax.fori_loop(0, _SUBSTEPS, step, 0)

    pltpu.sync_copy(x_v, out_hbm.at[1, r, pl.ds(j0, _CHUNK)])


def kernel(t, x, embed_table, wq, bq, wk, bk):
    B, D = x.shape
    ep = jnp.concatenate(
        [jnp.zeros((D, 1), _F32), embed_table[1:D + 1]], axis=1).reshape(-1)
    h = (t[1] - t[0]) / _SUBSTEPS
    cst = jnp.concatenate([wq[0], wk[0], jnp.broadcast_to(h, (16,))])

    mesh = plsc.VectorSubcoreMesh(core_axis_name="c", subcore_axis_name="s",
                                  num_cores=2, num_subcores=16)
    run = functools.partial(
        pl.kernel,
        out_type=jax.ShapeDtypeStruct((2, B, D), _F32),
        mesh=mesh,
        compiler_params=pltpu.CompilerParams(needs_layout_passes=False),
        scratch_types=[
            pltpu.VMEM((_CHUNK * 16,), _F32),
            pltpu.VMEM((_CHUNK,), _F32),
            pltpu.VMEM((_CHUNK,), _F32),
            pltpu.VMEM((_CHUNK,), _F32),
            pltpu.VMEM((_CHUNK * 16,), _F32),
            pltpu.VMEM((_CHUNK * 16,), _F32),
            pltpu.VMEM((16, _CHUNK), _F32),
            pltpu.VMEM((256,), _F32),
            pltpu.VMEM((256,), _F32),
            pltpu.VMEM((16,), _F32),
            pltpu.VMEM((16,), _F32),
            pltpu.VMEM((48,), _F32),
            pltpu.VMEM((128,), _F32),
            pltpu.VMEM((_TPR, 128), _F32),
            pltpu.VMEM_SHARED((2, _TPR, 128), _F32),
        ],
    )(_sc_body)
    return run(x, ep, wq.reshape(-1), wk.reshape(-1), bq, bk, cst)


# speedup vs baseline: 1.1154x; 1.1154x over previous
import functools
import jax
import jax.numpy as jnp
from jax import lax
from jax.experimental import pallas as pl
from jax.experimental.pallas import tpu as pltpu
from jax.experimental.pallas import tpu_sc as plsc

_D = 2048
_B = 4
_QK_SCALE = 16 ** -0.5
_SUBSTEPS = 8
_TPR = 8
_CHUNK = _D // _TPR
_NCH = _CHUNK // 16
_F32 = jnp.float32


def _bcast(v, lane):
    return jnp.take_along_axis(v, jnp.full((16,), lane, jnp.int32), axis=0)


def _hsum(v):
    return jnp.sum(v)


def _sc_body(x_hbm, ep_hbm, ept_hbm, wk_hbm, wqt_hbm, bq_hbm, bk_hbm, cst_hbm,
             out_hbm,
             ep_v, ept_v, x_v, xs_v, acc_v,
             wk_v, wqt_v, bq_v, bk_v, cst_v, part_v, all_v, accum_sh):
    cid = lax.axis_index("c")
    sid = lax.axis_index("s")
    row_l = sid // _TPR
    p = sid % _TPR
    r = cid * 2 + row_l
    j0 = p * _CHUNK

    pltpu.sync_copy(x_hbm.at[r, pl.ds(j0, _CHUNK)], x_v)
    pltpu.sync_copy(ep_hbm.at[pl.ds(j0 * 16, _CHUNK * 16)], ep_v)
    pltpu.sync_copy(ept_hbm.at[p], ept_v)
    pltpu.sync_copy(wk_hbm, wk_v)
    pltpu.sync_copy(wqt_hbm, wqt_v)
    pltpu.sync_copy(bq_hbm, bq_v)
    pltpu.sync_copy(bk_hbm, bk_v)
    pltpu.sync_copy(cst_hbm, cst_v)
    pltpu.sync_copy(x_v, out_hbm.at[0, r, pl.ds(j0, _CHUNK)])

    def init_xs(c, _):
        sl = pl.ds(c * 16, 16)
        xs_v[sl] = x_v[sl]
        return 0

    lax.fori_loop(0, _NCH, init_xs, 0)
    zv = jnp.zeros((16,), _F32)
    for i in range(3, 8):
        part_v[pl.ds(i * 16, 16)] = zv

    wq0 = cst_v[pl.ds(0, 16)]
    wk0 = cst_v[pl.ds(16, 16)]
    hv = cst_v[pl.ds(32, 16)]
    bqv = bq_v[...]
    bkv = bk_v[...]

    def do_stage(stage_i, par):
        def chunk_a(c, carry):
            ue, ssv, sxv = carry
            xc = xs_v[pl.ds(c * 16, 16)]
            ssv = ssv + xc * xc
            sxv = sxv + xc
            for l in range(16):
                bx = _bcast(xc, l)
                ue = ue + bx * ep_v[pl.ds((c * 16 + l) * 16, 16)]
            return ue, ssv, sxv

        z = jnp.zeros((16,), _F32)
        ue, ssv, sxv = lax.fori_loop(0, _NCH, chunk_a, (z, z, z))
        part_v[pl.ds(0, 16)] = ue
        part_v[pl.ds(16, 16)] = ssv
        part_v[pl.ds(32, 16)] = sxv

        pltpu.sync_copy(part_v, accum_sh.at[row_l, par, p])
        plsc.subcore_barrier()
        pltpu.sync_copy(accum_sh.at[row_l, par], all_v)

        uet = all_v[0, pl.ds(0, 16)]
        ssw = all_v[0, pl.ds(16, 16)]
        sxw = all_v[0, pl.ds(32, 16)]
        for q in range(1, _TPR):
            uet = uet + all_v[q, pl.ds(0, 16)]
            ssw = ssw + all_v[q, pl.ds(16, 16)]
            sxw = sxw + all_v[q, pl.ds(32, 16)]
        ss = _hsum(ssw)
        sx = _hsum(sxw)
        uk = sx * bkv
        for e in range(16):
            uk = uk + _bcast(uet, e) * wk_v[pl.ds(e * 16, 16)]
        m = wk0 * ss + uk
        a = _hsum(wq0 * m)
        mq = jnp.zeros((16,), _F32)
        for d in range(16):
            mq = mq + _bcast(m, d) * wqt_v[pl.ds(d * 16, 16)]
        bqm = _hsum(bqv * m)
        g = _QK_SCALE * (a * ss + _hsum(uet * mq) + sx * bqm)
        sbqm = _QK_SCALE * bqm
        mqb = [_QK_SCALE * _bcast(mq, e) for e in range(16)]
        sa = _QK_SCALE * a

        def chunk_d(c, _):
            sl = pl.ds(c * 16, 16)
            xsc = xs_v[sl]
            pf = mqb[0] * ept_v[0, sl]
            for d in range(1, 16):
                pf = pf + mqb[d] * ept_v[d, sl]
            fit = sa * xsc + pf + sbqm
            kc = xsc * (fit - g)
            if stage_i == 0:
                acc_v[sl] = kc
                xs_v[sl] = x_v[sl] + (0.5 * hv) * kc
            elif stage_i == 1:
                acc_v[sl] = acc_v[sl] + 2.0 * kc
                xs_v[sl] = x_v[sl] + (0.5 * hv) * kc
            elif stage_i == 2:
                acc_v[sl] = acc_v[sl] + 2.0 * kc
                xs_v[sl] = x_v[sl] + hv * kc
            else:
                xn = x_v[sl] + (hv * (1.0 / 6.0)) * (acc_v[sl] + kc)
                x_v[sl] = xn
                xs_v[sl] = xn
            return 0

        lax.fori_loop(0, _NCH, chunk_d, 0)

    def step(_, carry):
        do_stage(0, 0)
        do_stage(1, 1)
        do_stage(2, 0)
        do_stage(3, 1)
        return carry

    lax.fori_loop(0, _SUBSTEPS, step, 0)

    pltpu.sync_copy(x_v, out_hbm.at[1, r, pl.ds(j0, _CHUNK)])


def kernel(t, x, embed_table, wq, bq, wk, bk):
    B, D = x.shape
    ep2 = jnp.concatenate(
        [jnp.zeros((D, 1), _F32), embed_table[1:D + 1]], axis=1)
    ep = ep2.reshape(-1)
    ept8 = jnp.stack([ep2.T[:, i * _CHUNK:(i + 1) * _CHUNK]
                      for i in range(_TPR)], axis=0)
    h = (t[1] - t[0]) / _SUBSTEPS
    cst = jnp.concatenate([wq[0], wk[0], jnp.broadcast_to(h, (16,))])

    mesh = plsc.VectorSubcoreMesh(core_axis_name="c", subcore_axis_name="s",
                                  num_cores=2, num_subcores=16)
    run = functools.partial(
        pl.kernel,
        out_type=jax.ShapeDtypeStruct((2, B, D), _F32),
        mesh=mesh,
        compiler_params=pltpu.CompilerParams(needs_layout_passes=False),
        scratch_types=[
            pltpu.VMEM((_CHUNK * 16,), _F32),
            pltpu.VMEM((16, _CHUNK), _F32),
            pltpu.VMEM((_CHUNK,), _F32),
            pltpu.VMEM((_CHUNK,), _F32),
            pltpu.VMEM((_CHUNK,), _F32),
            pltpu.VMEM((256,), _F32),
            pltpu.VMEM((256,), _F32),
            pltpu.VMEM((16,), _F32),
            pltpu.VMEM((16,), _F32),
            pltpu.VMEM((48,), _F32),
            pltpu.VMEM((128,), _F32),
            pltpu.VMEM((_TPR, 128), _F32),
            pltpu.VMEM_SHARED((2, 2, _TPR, 128), _F32),
        ],
    )(_sc_body)
    return run(x, ep, ept8, wk.reshape(-1), wq.T.reshape(-1), bq, bk, cst)
